# baseline (speedup 1.0000x reference)
import functools
import math

import jax
import jax.numpy as jnp
import numpy as np
from jax.experimental import pallas as pl

_ACTION_SIZES = (5, 2, 4, 3, 2, 9, 2, 32, 35, 7, 2, 21, 2, 3, 3)
_NSEG = len(_ACTION_SIZES)
_DISC = 132
_BB = 1024
_HALF_LOG_2PI = 0.5 * math.log(2.0 * math.pi)


def _np_constants():
    S = np.zeros((_DISC, _NSEG), dtype=np.float32)
    starts = np.cumsum([0] + list(_ACTION_SIZES))
    for s, (c0, c1) in enumerate(zip(starts[:-1], starts[1:])):
        S[c0:c1, s] = 1.0
    E_cont = np.zeros((155, 23), dtype=np.float32)
    for i in range(23):
        E_cont[132 + i, i] = 1.0
    E_agent = np.zeros((161, 6), dtype=np.float32)
    for i in range(6):
        E_agent[155 + i, i] = 1.0
    return S, E_cont, E_agent, starts


_S_NP, _ECONT_NP, _EAGENT_NP, _STARTS = _np_constants()
_ST_NP = np.ascontiguousarray(_S_NP.T)


def _policy_kernel(state_ref, action_ref, w1t_ref, b1_ref, wdt_ref, bd_ref,
                   wcmt_ref, bcm_ref, wcst_ref, bcs_ref, wa1t_ref, ba1_ref,
                   wamt_ref, bam_ref, wast_ref, bas_ref, s_ref, st_ref,
                   econt_ref, eagent_ref, out_ref):
    f32 = jnp.float32
    bf16 = jnp.bfloat16
    x = state_ref[...]
    xb = x.astype(bf16)
    act = action_ref[...]

    h = jnp.dot(xb, w1t_ref[...], preferred_element_type=f32) + b1_ref[...]
    h = jnp.where(h >= 0.0, h, 0.01 * h)
    hb = h.astype(bf16)

    logits = jnp.dot(hb, wdt_ref[...], preferred_element_type=f32) + bd_ref[...]
    mean = jnp.clip(jnp.dot(h, wcmt_ref[...], preferred_element_type=f32)
                    + bcm_ref[...], -1.0, 1.0)
    logstd = jnp.clip(jnp.dot(h, wcst_ref[...], preferred_element_type=f32)
                      + bcs_ref[...], 0.0, 1.0)

    disc = act[:, :_DISC]
    continuous = jnp.dot(act, econt_ref[...], preferred_element_type=f32)

    gmax = jnp.max(logits, axis=1, keepdims=True)
    e = jnp.exp(logits - gmax)
    segsum = jnp.dot(e, s_ref[...], preferred_element_type=f32)
    lse = jnp.log(segsum) + gmax

    iota = jax.lax.broadcasted_iota(jnp.int32, logits.shape, 1)
    cols = []
    for s in range(_NSEG):
        c0, c1 = int(_STARTS[s]), int(_STARTS[s + 1])
        inseg = (iota >= c0) & (iota < c1)
        masked = jnp.where(inseg, disc, -jnp.inf)
        cols.append(jnp.argmax(masked, axis=1)[:, None].astype(f32))
    idx_all = jnp.concatenate(cols, axis=1)
    idxfull = jnp.dot(idx_all, st_ref[...], preferred_element_type=f32)
    oh = (iota.astype(f32) == idxfull).astype(f32)
    chosen = jnp.dot(logits * oh, s_ref[...], preferred_element_type=f32)
    seg_lp = chosen - lse

    cont_lp = (-(continuous - mean) ** 2 * (0.5 * jnp.exp(-2.0 * logstd))
               - logstd - _HALF_LOG_2PI)

    h2 = jnp.dot(xb, wa1t_ref[...], preferred_element_type=f32) + ba1_ref[...]
    h2 = jnp.where(h2 >= 0.0, h2, 0.01 * h2)
    m2 = jnp.clip(jnp.dot(h2, wamt_ref[...], preferred_element_type=f32)
                  + bam_ref[...], -1.0, 1.0)
    ls2 = jnp.clip(jnp.dot(h2, wast_ref[...], preferred_element_type=f32)
                   + bas_ref[...], 0.0, 1.0)
    aact = jnp.dot(x, eagent_ref[...], preferred_element_type=f32)
    agent_lp = (-(aact - m2) ** 2 * (0.5 * jnp.exp(-2.0 * ls2))
                - ls2 - _HALF_LOG_2PI)

    out_ref[...] = jnp.concatenate([seg_lp, cont_lp, agent_lp], axis=1)


@functools.partial(jax.jit, static_argnames=("interpret",))
def _run(state, action, W1, b1, Wd, bd, Wc, bc, Wa1, ba1, Wa2, ba2,
         interpret=False):
    B = state.shape[0]
    w1t = W1.T.astype(jnp.bfloat16)
    bd2 = bd[None, :]
    wdt = Wd.T.astype(jnp.bfloat16)
    wcmt = Wc[:23].T
    bcm = bc[None, :23]
    wcst = Wc[23:].T
    bcs = bc[None, 23:]
    wa1t = jnp.zeros((161, 128), jnp.bfloat16).at[:155, :].set(
        Wa1.T.astype(jnp.bfloat16))
    wamt = Wa2[:6].T
    bam = ba2[None, :6]
    wast = Wa2[6:].T
    bas = ba2[None, 6:]
    S = jnp.asarray(_S_NP)
    St = jnp.asarray(_ST_NP)
    econt = jnp.asarray(_ECONT_NP)
    eagent = jnp.asarray(_EAGENT_NP)

    grid = (B // _BB,)
    row = lambda i: (i, 0)
    rep = lambda i: (0, 0)
    full = lambda a: pl.BlockSpec(a.shape, rep)
    out = pl.pallas_call(
        _policy_kernel,
        grid=grid,
        in_specs=[
            pl.BlockSpec((_BB, 161), row),
            pl.BlockSpec((_BB, 155), row),
            full(w1t), full(b1[None, :]), full(wdt), full(bd2),
            full(wcmt), full(bcm), full(wcst), full(bcs),
            full(wa1t), full(ba1[None, :]),
            full(wamt), full(bam), full(wast), full(bas),
            full(S), full(St), full(econt), full(eagent),
        ],
        out_specs=pl.BlockSpec((_BB, 44), row),
        out_shape=jax.ShapeDtypeStruct((B, 44), jnp.float32),
        interpret=interpret,
    )(state, action, w1t, b1[None, :], wdt, bd2, wcmt, bcm, wcst, bcs,
      wa1t, ba1[None, :], wamt, bam, wast, bas, S, St, econt, eagent)
    return out


def kernel(state, action, W1, b1, Wd, bd, Wc, bc, Wa1, ba1, Wa2, ba2):
    return _run(state, action, W1, b1, Wd, bd, Wc, bc, Wa1, ba1, Wa2, ba2)


# speedup vs baseline: 1.3057x; 1.3057x over previous
import functools
import math

import jax
import jax.numpy as jnp
import numpy as np
from jax.experimental import pallas as pl
from jax.experimental.pallas import tpu as pltpu

_ACTION_SIZES = (5, 2, 4, 3, 2, 9, 2, 32, 35, 7, 2, 21, 2, 3, 3)
_NSEG = len(_ACTION_SIZES)
_DISC = 132
_BB = 1024
_HALF_LOG_2PI = 0.5 * math.log(2.0 * math.pi)


def _np_constants():
    S = np.zeros((_DISC, _NSEG), dtype=np.float32)
    starts = np.cumsum([0] + list(_ACTION_SIZES))
    for s, (c0, c1) in enumerate(zip(starts[:-1], starts[1:])):
        S[c0:c1, s] = 1.0
    E_cont = np.zeros((155, 23), dtype=np.float32)
    for i in range(23):
        E_cont[132 + i, i] = 1.0
    E_agent = np.zeros((161, 6), dtype=np.float32)
    for i in range(6):
        E_agent[155 + i, i] = 1.0
    return S, E_cont, E_agent, starts


_S_NP, _ECONT_NP, _EAGENT_NP, _STARTS = _np_constants()
_ST_NP = np.ascontiguousarray(_S_NP.T)


def _policy_kernel(state_ref, action_ref, w1t_ref, b1_ref, wdt_ref, bd_ref,
                   wcmt_ref, bcm_ref, wcst_ref, bcs_ref, wa1t_ref, ba1_ref,
                   wamt_ref, bam_ref, wast_ref, bas_ref, s_ref, st_ref,
                   econt_ref, eagent_ref, out_ref):
    f32 = jnp.float32
    bf16 = jnp.bfloat16
    x = state_ref[...]
    xb = x.astype(bf16)
    act = action_ref[...]

    h = jnp.dot(xb, w1t_ref[...], preferred_element_type=f32) + b1_ref[...]
    h = jnp.where(h >= 0.0, h, 0.01 * h)
    hb = h.astype(bf16)

    logits = jnp.dot(hb, wdt_ref[...], preferred_element_type=f32) + bd_ref[...]
    mean = jnp.clip(jnp.dot(h, wcmt_ref[...], preferred_element_type=f32)
                    + bcm_ref[...], -1.0, 1.0)
    logstd = jnp.clip(jnp.dot(h, wcst_ref[...], preferred_element_type=f32)
                      + bcs_ref[...], 0.0, 1.0)

    disc = act[:, :_DISC]
    continuous = jnp.dot(act, econt_ref[...], preferred_element_type=f32)

    gmax = jnp.max(logits, axis=1, keepdims=True)
    e = jnp.exp(logits - gmax)
    segsum = jnp.dot(e, s_ref[...], preferred_element_type=f32)
    lse = jnp.log(segsum) + gmax

    iota = jax.lax.broadcasted_iota(jnp.int32, logits.shape, 1)
    oh = jnp.zeros(logits.shape, dtype=f32)
    for s in range(_NSEG):
        c0, c1 = int(_STARTS[s]), int(_STARTS[s + 1])
        idx = jnp.argmax(disc[:, c0:c1], axis=1)[:, None] + c0
        oh = oh + (iota == idx).astype(f32)
    chosen = jnp.dot(logits * oh, s_ref[...], preferred_element_type=f32)
    seg_lp = chosen - lse

    cont_lp = (-(continuous - mean) ** 2 * (0.5 * jnp.exp(-2.0 * logstd))
               - logstd - _HALF_LOG_2PI)

    h2 = jnp.dot(xb, wa1t_ref[...], preferred_element_type=f32) + ba1_ref[...]
    h2 = jnp.where(h2 >= 0.0, h2, 0.01 * h2)
    m2 = jnp.clip(jnp.dot(h2, wamt_ref[...], preferred_element_type=f32)
                  + bam_ref[...], -1.0, 1.0)
    ls2 = jnp.clip(jnp.dot(h2, wast_ref[...], preferred_element_type=f32)
                   + bas_ref[...], 0.0, 1.0)
    aact = jnp.dot(x, eagent_ref[...], preferred_element_type=f32)
    agent_lp = (-(aact - m2) ** 2 * (0.5 * jnp.exp(-2.0 * ls2))
                - ls2 - _HALF_LOG_2PI)

    out_ref[...] = jnp.concatenate([seg_lp, cont_lp, agent_lp], axis=1)


@functools.partial(jax.jit, static_argnames=("interpret",))
def _run(state, action, W1, b1, Wd, bd, Wc, bc, Wa1, ba1, Wa2, ba2,
         interpret=False):
    B = state.shape[0]
    w1t = W1.T.astype(jnp.bfloat16)
    bd2 = bd[None, :]
    wdt = Wd.T.astype(jnp.bfloat16)
    wcmt = Wc[:23].T
    bcm = bc[None, :23]
    wcst = Wc[23:].T
    bcs = bc[None, 23:]
    wa1t = jnp.zeros((161, 128), jnp.bfloat16).at[:155, :].set(
        Wa1.T.astype(jnp.bfloat16))
    wamt = Wa2[:6].T
    bam = ba2[None, :6]
    wast = Wa2[6:].T
    bas = ba2[None, 6:]
    S = jnp.asarray(_S_NP)
    St = jnp.asarray(_ST_NP)
    econt = jnp.asarray(_ECONT_NP)
    eagent = jnp.asarray(_EAGENT_NP)

    grid = (B // _BB,)
    row = lambda i: (i, 0)
    rep = lambda i: (0, 0)
    full = lambda a: pl.BlockSpec(a.shape, rep)
    out = pl.pallas_call(
        _policy_kernel,
        grid=grid,
        in_specs=[
            pl.BlockSpec((_BB, 161), row),
            pl.BlockSpec((_BB, 155), row),
            full(w1t), full(b1[None, :]), full(wdt), full(bd2),
            full(wcmt), full(bcm), full(wcst), full(bcs),
            full(wa1t), full(ba1[None, :]),
            full(wamt), full(bam), full(wast), full(bas),
            full(S), full(St), full(econt), full(eagent),
        ],
        out_specs=pl.BlockSpec((_BB, 44), row),
        out_shape=jax.ShapeDtypeStruct((B, 44), jnp.float32),
        compiler_params=pltpu.CompilerParams(
            dimension_semantics=("parallel",)),
        interpret=interpret,
    )(state, action, w1t, b1[None, :], wdt, bd2, wcmt, bcm, wcst, bcs,
      wa1t, ba1[None, :], wamt, bam, wast, bas, S, St, econt, eagent)
    return out


def kernel(state, action, W1, b1, Wd, bd, Wc, bc, Wa1, ba1, Wa2, ba2):
    return _run(state, action, W1, b1, Wd, bd, Wc, bc, Wa1, ba1, Wa2, ba2)


# speedup vs baseline: 1.3227x; 1.0130x over previous
import functools
import math

import jax
import jax.numpy as jnp
import numpy as np
from jax.experimental import pallas as pl
from jax.experimental.pallas import tpu as pltpu

_ACTION_SIZES = (5, 2, 4, 3, 2, 9, 2, 32, 35, 7, 2, 21, 2, 3, 3)
_NSEG = len(_ACTION_SIZES)
_DISC = 132
_BB = 2048
_HALF_LOG_2PI = 0.5 * math.log(2.0 * math.pi)


def _np_constants():
    S = np.zeros((_DISC, _NSEG), dtype=np.float32)
    starts = np.cumsum([0] + list(_ACTION_SIZES))
    for s, (c0, c1) in enumerate(zip(starts[:-1], starts[1:])):
        S[c0:c1, s] = 1.0
    E_cont = np.zeros((155, 23), dtype=np.float32)
    for i in range(23):
        E_cont[132 + i, i] = 1.0
    E_agent = np.zeros((161, 6), dtype=np.float32)
    for i in range(6):
        E_agent[155 + i, i] = 1.0
    return S, E_cont, E_agent, starts


_S_NP, _ECONT_NP, _EAGENT_NP, _STARTS = _np_constants()
_ST_NP = np.ascontiguousarray(_S_NP.T)


def _policy_kernel(state_ref, action_ref, w1t_ref, b1_ref, wdt_ref, bd_ref,
                   wcmt_ref, bcm_ref, wcst_ref, bcs_ref, wa1t_ref, ba1_ref,
                   wamt_ref, bam_ref, wast_ref, bas_ref, s_ref, st_ref,
                   econt_ref, eagent_ref, out_ref):
    f32 = jnp.float32
    bf16 = jnp.bfloat16
    x = state_ref[...]
    xb = x.astype(bf16)
    act = action_ref[...]

    h = jnp.dot(xb, w1t_ref[...], preferred_element_type=f32) + b1_ref[...]
    h = jnp.where(h >= 0.0, h, 0.01 * h)
    hb = h.astype(bf16)

    logits = jnp.dot(hb, wdt_ref[...], preferred_element_type=f32) + bd_ref[...]
    mean = jnp.clip(jnp.dot(h, wcmt_ref[...], preferred_element_type=f32)
                    + bcm_ref[...], -1.0, 1.0)
    logstd = jnp.clip(jnp.dot(h, wcst_ref[...], preferred_element_type=f32)
                      + bcs_ref[...], 0.0, 1.0)

    disc = act[:, :_DISC]
    continuous = jnp.dot(act, econt_ref[...], preferred_element_type=f32)

    gmax = jnp.max(logits, axis=1, keepdims=True)
    e = jnp.exp(logits - gmax)
    segsum = jnp.dot(e, s_ref[...], preferred_element_type=f32)
    lse = jnp.log(segsum) + gmax

    iota = jax.lax.broadcasted_iota(jnp.int32, logits.shape, 1)
    oh = jnp.zeros(logits.shape, dtype=f32)
    for s in range(_NSEG):
        c0, c1 = int(_STARTS[s]), int(_STARTS[s + 1])
        idx = jnp.argmax(disc[:, c0:c1], axis=1)[:, None] + c0
        oh = oh + (iota == idx).astype(f32)
    chosen = jnp.dot(logits * oh, s_ref[...], preferred_element_type=f32)
    seg_lp = chosen - lse

    cont_lp = (-(continuous - mean) ** 2 * (0.5 * jnp.exp(-2.0 * logstd))
               - logstd - _HALF_LOG_2PI)

    h2 = jnp.dot(xb, wa1t_ref[...], preferred_element_type=f32) + ba1_ref[...]
    h2 = jnp.where(h2 >= 0.0, h2, 0.01 * h2)
    m2 = jnp.clip(jnp.dot(h2, wamt_ref[...], preferred_element_type=f32)
                  + bam_ref[...], -1.0, 1.0)
    ls2 = jnp.clip(jnp.dot(h2, wast_ref[...], preferred_element_type=f32)
                   + bas_ref[...], 0.0, 1.0)
    aact = jnp.dot(x, eagent_ref[...], preferred_element_type=f32)
    agent_lp = (-(aact - m2) ** 2 * (0.5 * jnp.exp(-2.0 * ls2))
                - ls2 - _HALF_LOG_2PI)

    out_ref[...] = jnp.concatenate([seg_lp, cont_lp, agent_lp], axis=1)


@functools.partial(jax.jit, static_argnames=("interpret",))
def _run(state, action, W1, b1, Wd, bd, Wc, bc, Wa1, ba1, Wa2, ba2,
         interpret=False):
    B = state.shape[0]
    w1t = W1.T.astype(jnp.bfloat16)
    bd2 = bd[None, :]
    wdt = Wd.T.astype(jnp.bfloat16)
    wcmt = Wc[:23].T
    bcm = bc[None, :23]
    wcst = Wc[23:].T
    bcs = bc[None, 23:]
    wa1t = jnp.zeros((161, 128), jnp.bfloat16).at[:155, :].set(
        Wa1.T.astype(jnp.bfloat16))
    wamt = Wa2[:6].T
    bam = ba2[None, :6]
    wast = Wa2[6:].T
    bas = ba2[None, 6:]
    S = jnp.asarray(_S_NP)
    St = jnp.asarray(_ST_NP)
    econt = jnp.asarray(_ECONT_NP)
    eagent = jnp.asarray(_EAGENT_NP)

    grid = (B // _BB,)
    row = lambda i: (i, 0)
    rep = lambda i: (0, 0)
    full = lambda a: pl.BlockSpec(a.shape, rep)
    out = pl.pallas_call(
        _policy_kernel,
        grid=grid,
        in_specs=[
            pl.BlockSpec((_BB, 161), row),
            pl.BlockSpec((_BB, 155), row),
            full(w1t), full(b1[None, :]), full(wdt), full(bd2),
            full(wcmt), full(bcm), full(wcst), full(bcs),
            full(wa1t), full(ba1[None, :]),
            full(wamt), full(bam), full(wast), full(bas),
            full(S), full(St), full(econt), full(eagent),
        ],
        out_specs=pl.BlockSpec((_BB, 44), row),
        out_shape=jax.ShapeDtypeStruct((B, 44), jnp.float32),
        compiler_params=pltpu.CompilerParams(
            dimension_semantics=("parallel",)),
        interpret=interpret,
    )(state, action, w1t, b1[None, :], wdt, bd2, wcmt, bcm, wcst, bcs,
      wa1t, ba1[None, :], wamt, bam, wast, bas, S, St, econt, eagent)
    return out


def kernel(state, action, W1, b1, Wd, bd, Wc, bc, Wa1, ba1, Wa2, ba2):
    return _run(state, action, W1, b1, Wd, bd, Wc, bc, Wa1, ba1, Wa2, ba2)


# speedup vs baseline: 1.3325x; 1.0074x over previous
import functools
import math

import jax
import jax.numpy as jnp
import numpy as np
from jax.experimental import pallas as pl
from jax.experimental.pallas import tpu as pltpu

_ACTION_SIZES = (5, 2, 4, 3, 2, 9, 2, 32, 35, 7, 2, 21, 2, 3, 3)
_NSEG = len(_ACTION_SIZES)
_DISC = 132
_BB = 2048
_HALF_LOG_2PI = 0.5 * math.log(2.0 * math.pi)


def _np_constants():
    S = np.zeros((_DISC, _NSEG), dtype=np.float32)
    starts = np.cumsum([0] + list(_ACTION_SIZES))
    for s, (c0, c1) in enumerate(zip(starts[:-1], starts[1:])):
        S[c0:c1, s] = 1.0
    E_cont = np.zeros((155, 23), dtype=np.float32)
    for i in range(23):
        E_cont[132 + i, i] = 1.0
    E_agent = np.zeros((161, 6), dtype=np.float32)
    for i in range(6):
        E_agent[155 + i, i] = 1.0
    return S, E_cont, E_agent, starts


_S_NP, _ECONT_NP, _EAGENT_NP, _STARTS = _np_constants()
_ST_NP = np.ascontiguousarray(_S_NP.T)


def _policy_kernel(state_ref, action_ref, w1t_ref, b1_ref, wdt_ref, bd_ref,
                   wcmt_ref, bcm_ref, wcst_ref, bcs_ref, wa1t_ref, ba1_ref,
                   wamt_ref, bam_ref, wast_ref, bas_ref, s_ref, st_ref,
                   econt_ref, eagent_ref, out_ref):
    f32 = jnp.float32
    bf16 = jnp.bfloat16
    x = state_ref[...]
    xb = x.astype(bf16)
    act = action_ref[...]

    h = jnp.dot(xb, w1t_ref[...], preferred_element_type=f32) + b1_ref[...]
    h = jnp.where(h >= 0.0, h, 0.01 * h)
    hb = h.astype(bf16)

    logits = jnp.dot(hb, wdt_ref[...], preferred_element_type=f32) + bd_ref[...]
    mean = jnp.clip(jnp.dot(h, wcmt_ref[...], preferred_element_type=f32)
                    + bcm_ref[...], -1.0, 1.0)
    logstd = jnp.clip(jnp.dot(h, wcst_ref[...], preferred_element_type=f32)
                      + bcs_ref[...], 0.0, 1.0)

    disc = act[:, :_DISC]
    continuous = jnp.dot(act, econt_ref[...], preferred_element_type=f32)

    gmax = jnp.max(logits, axis=1, keepdims=True)
    e = jnp.exp(logits - gmax)
    segsum = jnp.dot(e, s_ref[...], preferred_element_type=f32)
    lse = jnp.log(segsum) + gmax

    bb = logits.shape[0]
    iota_lo = jax.lax.broadcasted_iota(jnp.int32, (bb, 128), 1)
    iota_hi = jax.lax.broadcasted_iota(jnp.int32, (bb, _DISC - 128), 1) + 128
    oh_lo = jnp.zeros((bb, 128), dtype=f32)
    oh_hi = jnp.zeros((bb, _DISC - 128), dtype=f32)
    for s in range(_NSEG):
        c0, c1 = int(_STARTS[s]), int(_STARTS[s + 1])
        idx = jnp.argmax(disc[:, c0:c1], axis=1)[:, None] + c0
        if c0 < 128:
            oh_lo = oh_lo + (iota_lo == idx).astype(f32)
        if c1 > 128:
            oh_hi = oh_hi + (iota_hi == idx).astype(f32)
    oh = jnp.concatenate([oh_lo, oh_hi], axis=1)
    chosen = jnp.dot(logits * oh, s_ref[...], preferred_element_type=f32)
    seg_lp = chosen - lse

    cont_lp = (-(continuous - mean) ** 2 * (0.5 * jnp.exp(-2.0 * logstd))
               - logstd - _HALF_LOG_2PI)

    h2 = jnp.dot(xb, wa1t_ref[...], preferred_element_type=f32) + ba1_ref[...]
    h2 = jnp.where(h2 >= 0.0, h2, 0.01 * h2)
    m2 = jnp.clip(jnp.dot(h2, wamt_ref[...], preferred_element_type=f32)
                  + bam_ref[...], -1.0, 1.0)
    ls2 = jnp.clip(jnp.dot(h2, wast_ref[...], preferred_element_type=f32)
                   + bas_ref[...], 0.0, 1.0)
    aact = jnp.dot(x, eagent_ref[...], preferred_element_type=f32)
    agent_lp = (-(aact - m2) ** 2 * (0.5 * jnp.exp(-2.0 * ls2))
                - ls2 - _HALF_LOG_2PI)

    out_ref[...] = jnp.concatenate([seg_lp, cont_lp, agent_lp], axis=1)


@functools.partial(jax.jit, static_argnames=("interpret",))
def _run(state, action, W1, b1, Wd, bd, Wc, bc, Wa1, ba1, Wa2, ba2,
         interpret=False):
    B = state.shape[0]
    w1t = W1.T.astype(jnp.bfloat16)
    bd2 = bd[None, :]
    wdt = Wd.T.astype(jnp.bfloat16)
    wcmt = Wc[:23].T
    bcm = bc[None, :23]
    wcst = Wc[23:].T
    bcs = bc[None, 23:]
    wa1t = jnp.zeros((161, 128), jnp.bfloat16).at[:155, :].set(
        Wa1.T.astype(jnp.bfloat16))
    wamt = Wa2[:6].T
    bam = ba2[None, :6]
    wast = Wa2[6:].T
    bas = ba2[None, 6:]
    S = jnp.asarray(_S_NP)
    St = jnp.asarray(_ST_NP)
    econt = jnp.asarray(_ECONT_NP)
    eagent = jnp.asarray(_EAGENT_NP)

    grid = (B // _BB,)
    row = lambda i: (i, 0)
    rep = lambda i: (0, 0)
    full = lambda a: pl.BlockSpec(a.shape, rep)
    out = pl.pallas_call(
        _policy_kernel,
        grid=grid,
        in_specs=[
            pl.BlockSpec((_BB, 161), row),
            pl.BlockSpec((_BB, 155), row),
            full(w1t), full(b1[None, :]), full(wdt), full(bd2),
            full(wcmt), full(bcm), full(wcst), full(bcs),
            full(wa1t), full(ba1[None, :]),
            full(wamt), full(bam), full(wast), full(bas),
            full(S), full(St), full(econt), full(eagent),
        ],
        out_specs=pl.BlockSpec((_BB, 44), row),
        out_shape=jax.ShapeDtypeStruct((B, 44), jnp.float32),
        compiler_params=pltpu.CompilerParams(
            dimension_semantics=("parallel",)),
        interpret=interpret,
    )(state, action, w1t, b1[None, :], wdt, bd2, wcmt, bcm, wcst, bcs,
      wa1t, ba1[None, :], wamt, bam, wast, bas, S, St, econt, eagent)
    return out


def kernel(state, action, W1, b1, Wd, bd, Wc, bc, Wa1, ba1, Wa2, ba2):
    return _run(state, action, W1, b1, Wd, bd, Wc, bc, Wa1, ba1, Wa2, ba2)


# speedup vs baseline: 1.4799x; 1.1106x over previous
import functools
import math

import jax
import jax.numpy as jnp
import numpy as np
from jax.experimental import pallas as pl
from jax.experimental.pallas import tpu as pltpu

_ACTION_SIZES = (5, 2, 4, 3, 2, 9, 2, 32, 35, 7, 2, 21, 2, 3, 3)
_NSEG = len(_ACTION_SIZES)
_DISC = 132
_BB = 2048
_HALF_LOG_2PI = 0.5 * math.log(2.0 * math.pi)


def _np_constants():
    S = np.zeros((_DISC, _NSEG), dtype=np.float32)
    starts = np.cumsum([0] + list(_ACTION_SIZES))
    for s, (c0, c1) in enumerate(zip(starts[:-1], starts[1:])):
        S[c0:c1, s] = 1.0
    E_cont = np.zeros((155, 23), dtype=np.float32)
    for i in range(23):
        E_cont[132 + i, i] = 1.0
    E_agent = np.zeros((161, 6), dtype=np.float32)
    for i in range(6):
        E_agent[155 + i, i] = 1.0
    return S, E_cont, E_agent, starts


_S_NP, _ECONT_NP, _EAGENT_NP, _STARTS = _np_constants()
_ST_NP = np.ascontiguousarray(_S_NP.T)


def _policy_kernel(state_ref, action_ref, w1t_ref, b1_ref, wdt_ref, bd_ref,
                   wcmt_ref, bcm_ref, wcst_ref, bcs_ref, wa1t_ref, ba1_ref,
                   wamt_ref, bam_ref, wast_ref, bas_ref, s_ref, st_ref,
                   econt_ref, eagent_ref, out_ref):
    f32 = jnp.float32
    bf16 = jnp.bfloat16
    x = state_ref[...]
    xb = x.astype(bf16)
    act = action_ref[...]
    disc = act[:, :_DISC]

    bb = disc.shape[0]
    iota_lo = jax.lax.broadcasted_iota(jnp.int32, (bb, 128), 1)
    iota_hi = jax.lax.broadcasted_iota(jnp.int32, (bb, _DISC - 128), 1) + 128
    oh_lo = jnp.zeros((bb, 128), dtype=f32)
    oh_hi = jnp.zeros((bb, _DISC - 128), dtype=f32)
    for s in range(_NSEG):
        c0, c1 = int(_STARTS[s]), int(_STARTS[s + 1])
        idx = jnp.argmax(disc[:, c0:c1], axis=1)[:, None] + c0
        if c0 < 128:
            oh_lo = oh_lo + (iota_lo == idx).astype(f32)
        if c1 > 128:
            oh_hi = oh_hi + (iota_hi == idx).astype(f32)
    oh = jnp.concatenate([oh_lo, oh_hi], axis=1)

    h = jnp.dot(xb, w1t_ref[...], preferred_element_type=f32) + b1_ref[...]
    h = jnp.where(h >= 0.0, h, 0.01 * h)
    hb = h.astype(bf16)

    logits = jnp.dot(hb, wdt_ref[...], preferred_element_type=f32) + bd_ref[...]
    mean = jnp.clip(jnp.dot(h, wcmt_ref[...], preferred_element_type=f32)
                    + bcm_ref[...], -1.0, 1.0)
    logstd = jnp.clip(jnp.dot(h, wcst_ref[...], preferred_element_type=f32)
                      + bcs_ref[...], 0.0, 1.0)

    continuous = jnp.dot(act, econt_ref[...], preferred_element_type=f32)

    gmax = jnp.max(logits, axis=1, keepdims=True)
    e = jnp.exp(logits - gmax)
    segsum = jnp.dot(e, s_ref[...], preferred_element_type=f32)
    lse = jnp.log(segsum) + gmax

    chosen = jnp.dot(logits * oh, s_ref[...], preferred_element_type=f32)
    seg_lp = chosen - lse

    cont_lp = (-(continuous - mean) ** 2 * (0.5 * jnp.exp(-2.0 * logstd))
               - logstd - _HALF_LOG_2PI)

    h2 = jnp.dot(xb, wa1t_ref[...], preferred_element_type=f32) + ba1_ref[...]
    h2 = jnp.where(h2 >= 0.0, h2, 0.01 * h2)
    m2 = jnp.clip(jnp.dot(h2, wamt_ref[...], preferred_element_type=f32)
                  + bam_ref[...], -1.0, 1.0)
    ls2 = jnp.clip(jnp.dot(h2, wast_ref[...], preferred_element_type=f32)
                   + bas_ref[...], 0.0, 1.0)
    aact = jnp.dot(x, eagent_ref[...], preferred_element_type=f32)
    agent_lp = (-(aact - m2) ** 2 * (0.5 * jnp.exp(-2.0 * ls2))
                - ls2 - _HALF_LOG_2PI)

    out_ref[...] = jnp.concatenate([seg_lp, cont_lp, agent_lp], axis=1)


@functools.partial(jax.jit, static_argnames=("interpret",))
def _run(state, action, W1, b1, Wd, bd, Wc, bc, Wa1, ba1, Wa2, ba2,
         interpret=False):
    B = state.shape[0]
    w1t = W1.T.astype(jnp.bfloat16)
    bd2 = bd[None, :]
    wdt = Wd.T.astype(jnp.bfloat16)
    wcmt = Wc[:23].T
    bcm = bc[None, :23]
    wcst = Wc[23:].T
    bcs = bc[None, 23:]
    wa1t = jnp.zeros((161, 128), jnp.bfloat16).at[:155, :].set(
        Wa1.T.astype(jnp.bfloat16))
    wamt = Wa2[:6].T
    bam = ba2[None, :6]
    wast = Wa2[6:].T
    bas = ba2[None, 6:]
    S = jnp.asarray(_S_NP)
    St = jnp.asarray(_ST_NP)
    econt = jnp.asarray(_ECONT_NP)
    eagent = jnp.asarray(_EAGENT_NP)

    grid = (B // _BB,)
    row = lambda i: (i, 0)
    rep = lambda i: (0, 0)
    full = lambda a: pl.BlockSpec(a.shape, rep)
    out = pl.pallas_call(
        _policy_kernel,
        grid=grid,
        in_specs=[
            pl.BlockSpec((_BB, 161), row),
            pl.BlockSpec((_BB, 155), row),
            full(w1t), full(b1[None, :]), full(wdt), full(bd2),
            full(wcmt), full(bcm), full(wcst), full(bcs),
            full(wa1t), full(ba1[None, :]),
            full(wamt), full(bam), full(wast), full(bas),
            full(S), full(St), full(econt), full(eagent),
        ],
        out_specs=pl.BlockSpec((_BB, 44), row),
        out_shape=jax.ShapeDtypeStruct((B, 44), jnp.float32),
        compiler_params=pltpu.CompilerParams(
            dimension_semantics=("parallel",)),
        interpret=interpret,
    )(state, action, w1t, b1[None, :], wdt, bd2, wcmt, bcm, wcst, bcs,
      wa1t, ba1[None, :], wamt, bam, wast, bas, S, St, econt, eagent)
    return out


def kernel(state, action, W1, b1, Wd, bd, Wc, bc, Wa1, ba1, Wa2, ba2):
    return _run(state, action, W1, b1, Wd, bd, Wc, bc, Wa1, ba1, Wa2, ba2)


# speedup vs baseline: 1.6003x; 1.0814x over previous
import functools
import math

import jax
import jax.numpy as jnp
import numpy as np
from jax import lax
from jax.experimental import pallas as pl
from jax.experimental.pallas import tpu as pltpu
from jax.experimental.pallas import tpu_sc as plsc

_ACTION_SIZES = (5, 2, 4, 3, 2, 9, 2, 32, 35, 7, 2, 21, 2, 3, 3)
_NSEG = len(_ACTION_SIZES)
_DISC = 132
_BB = 2048
_HALF_LOG_2PI = 0.5 * math.log(2.0 * math.pi)


def _np_constants():
    S = np.zeros((_DISC, _NSEG), dtype=np.float32)
    starts = np.cumsum([0] + list(_ACTION_SIZES))
    for s, (c0, c1) in enumerate(zip(starts[:-1], starts[1:])):
        S[c0:c1, s] = 1.0
    S16 = np.zeros((16, _DISC), dtype=np.float32)
    S16[:_NSEG, :] = S.T
    E_cont = np.zeros((155, 23), dtype=np.float32)
    for i in range(23):
        E_cont[132 + i, i] = 1.0
    E_agent = np.zeros((161, 6), dtype=np.float32)
    for i in range(6):
        E_agent[155 + i, i] = 1.0
    return S, S16, E_cont, E_agent, starts


_S_NP, _S16_NP, _ECONT_NP, _EAGENT_NP, _STARTS = _np_constants()


def _sc_argmax(action):
    B, ACT_W = action.shape
    NC, NS, L = 2, 16, 16
    NW = NC * NS
    rows_per_w = B // NW
    ntiles = rows_per_w // L

    @functools.partial(
        pl.kernel,
        out_type=jax.ShapeDtypeStruct((B, 16), jnp.float32),
        mesh=plsc.VectorSubcoreMesh(core_axis_name="c", subcore_axis_name="s"),
        scratch_types=[
            pltpu.VMEM((rows_per_w, ACT_W), jnp.float32),
            pltpu.VMEM((rows_per_w, 16), jnp.float32),
        ],
        compiler_params=pltpu.CompilerParams(use_tc_tiling_on_sc=False,
                                             needs_layout_passes=False),
    )
    def k(action_hbm, out_hbm, abuf, obuf):
        wid = lax.axis_index("s") * NC + lax.axis_index("c")
        base = wid * rows_per_w
        pltpu.sync_copy(action_hbm.at[pl.ds(base, rows_per_w)], abuf)
        lanes = lax.iota(jnp.int32, L)

        def tile_body(t, carry):
            rows = t * L + lanes
            zero = jnp.zeros((L,), jnp.float32)
            for s in range(_NSEG):
                c0, c1 = int(_STARTS[s]), int(_STARTS[s + 1])
                m = plsc.load_gather(abuf, [rows, jnp.full((L,), c0, jnp.int32)])
                bi = zero + float(c0)
                for j in range(c0 + 1, c1):
                    v = plsc.load_gather(
                        abuf, [rows, jnp.full((L,), j, jnp.int32)])
                    better = v > m
                    m = jnp.where(better, v, m)
                    bi = jnp.where(better, zero + float(j), bi)
                plsc.store_scatter(obuf, [rows, jnp.full((L,), s, jnp.int32)], bi)
            plsc.store_scatter(obuf, [rows, jnp.full((L,), 15, jnp.int32)], zero)
            return carry

        lax.fori_loop(0, ntiles, tile_body, 0)
        pltpu.sync_copy(obuf, out_hbm.at[pl.ds(base, rows_per_w)])

    return k(action)


def _policy_kernel(state_ref, action_ref, idx_ref, w1t_ref, b1_ref, wdt_ref,
                   bd_ref, wcmt_ref, bcm_ref, wcst_ref, bcs_ref, wa1t_ref,
                   ba1_ref, wamt_ref, bam_ref, wast_ref, bas_ref, s_ref,
                   s16_ref, econt_ref, eagent_ref, out_ref):
    f32 = jnp.float32
    bf16 = jnp.bfloat16
    x = state_ref[...]
    xb = x.astype(bf16)
    act = action_ref[...]

    idxfull = jnp.dot(idx_ref[...], s16_ref[...], preferred_element_type=f32)
    iota = jax.lax.broadcasted_iota(jnp.int32, (idxfull.shape[0], _DISC), 1)
    oh = (iota == idxfull.astype(jnp.int32)).astype(f32)

    h = jnp.dot(xb, w1t_ref[...], preferred_element_type=f32) + b1_ref[...]
    h = jnp.where(h >= 0.0, h, 0.01 * h)
    hb = h.astype(bf16)

    logits = jnp.dot(hb, wdt_ref[...], preferred_element_type=f32) + bd_ref[...]
    mean = jnp.clip(jnp.dot(h, wcmt_ref[...], preferred_element_type=f32)
                    + bcm_ref[...], -1.0, 1.0)
    logstd = jnp.clip(jnp.dot(h, wcst_ref[...], preferred_element_type=f32)
                      + bcs_ref[...], 0.0, 1.0)

    continuous = jnp.dot(act, econt_ref[...], preferred_element_type=f32)

    gmax = jnp.max(logits, axis=1, keepdims=True)
    e = jnp.exp(logits - gmax)
    segsum = jnp.dot(e, s_ref[...], preferred_element_type=f32)
    lse = jnp.log(segsum) + gmax

    chosen = jnp.dot(logits * oh, s_ref[...], preferred_element_type=f32)
    seg_lp = chosen - lse

    cont_lp = (-(continuous - mean) ** 2 * (0.5 * jnp.exp(-2.0 * logstd))
               - logstd - _HALF_LOG_2PI)

    h2 = jnp.dot(xb, wa1t_ref[...], preferred_element_type=f32) + ba1_ref[...]
    h2 = jnp.where(h2 >= 0.0, h2, 0.01 * h2)
    m2 = jnp.clip(jnp.dot(h2, wamt_ref[...], preferred_element_type=f32)
                  + bam_ref[...], -1.0, 1.0)
    ls2 = jnp.clip(jnp.dot(h2, wast_ref[...], preferred_element_type=f32)
                   + bas_ref[...], 0.0, 1.0)
    aact = jnp.dot(x, eagent_ref[...], preferred_element_type=f32)
    agent_lp = (-(aact - m2) ** 2 * (0.5 * jnp.exp(-2.0 * ls2))
                - ls2 - _HALF_LOG_2PI)

    out_ref[...] = jnp.concatenate([seg_lp, cont_lp, agent_lp], axis=1)


def _run_tc(state, action, idx16, W1, b1, Wd, bd, Wc, bc, Wa1, ba1, Wa2, ba2,
            interpret=False):
    B = state.shape[0]
    w1t = W1.T.astype(jnp.bfloat16)
    bd2 = bd[None, :]
    wdt = Wd.T.astype(jnp.bfloat16)
    wcmt = Wc[:23].T
    bcm = bc[None, :23]
    wcst = Wc[23:].T
    bcs = bc[None, 23:]
    wa1t = jnp.zeros((161, 128), jnp.bfloat16).at[:155, :].set(
        Wa1.T.astype(jnp.bfloat16))
    wamt = Wa2[:6].T
    bam = ba2[None, :6]
    wast = Wa2[6:].T
    bas = ba2[None, 6:]
    S = jnp.asarray(_S_NP)
    S16 = jnp.asarray(_S16_NP)
    econt = jnp.asarray(_ECONT_NP)
    eagent = jnp.asarray(_EAGENT_NP)

    grid = (B // _BB,)
    row = lambda i: (i, 0)
    rep = lambda i: (0, 0)
    full = lambda a: pl.BlockSpec(a.shape, rep)
    out = pl.pallas_call(
        _policy_kernel,
        grid=grid,
        in_specs=[
            pl.BlockSpec((_BB, 161), row),
            pl.BlockSpec((_BB, 155), row),
            pl.BlockSpec((_BB, 16), row),
            full(w1t), full(b1[None, :]), full(wdt), full(bd2),
            full(wcmt), full(bcm), full(wcst), full(bcs),
            full(wa1t), full(ba1[None, :]),
            full(wamt), full(bam), full(wast), full(bas),
            full(S), full(S16), full(econt), full(eagent),
        ],
        out_specs=pl.BlockSpec((_BB, 44), row),
        out_shape=jax.ShapeDtypeStruct((B, 44), jnp.float32),
        compiler_params=pltpu.CompilerParams(
            dimension_semantics=("parallel",)),
        interpret=interpret,
    )(state, action, idx16, w1t, b1[None, :], wdt, bd2, wcmt, bcm, wcst, bcs,
      wa1t, ba1[None, :], wamt, bam, wast, bas, S, S16, econt, eagent)
    return out


@jax.jit
def _run(state, action, W1, b1, Wd, bd, Wc, bc, Wa1, ba1, Wa2, ba2):
    idx16 = _sc_argmax(action)
    return _run_tc(state, action, idx16, W1, b1, Wd, bd, Wc, bc,
                   Wa1, ba1, Wa2, ba2)


def kernel(state, action, W1, b1, Wd, bd, Wc, bc, Wa1, ba1, Wa2, ba2):
    return _run(state, action, W1, b1, Wd, bd, Wc, bc, Wa1, ba1, Wa2, ba2)


# speedup vs baseline: 1.6067x; 1.0040x over previous
import functools
import math

import jax
import jax.numpy as jnp
import numpy as np
from jax import lax
from jax.experimental import pallas as pl
from jax.experimental.pallas import tpu as pltpu
from jax.experimental.pallas import tpu_sc as plsc

_ACTION_SIZES = (5, 2, 4, 3, 2, 9, 2, 32, 35, 7, 2, 21, 2, 3, 3)
_NSEG = len(_ACTION_SIZES)
_DISC = 132
_BB = 2048
_HALF_LOG_2PI = 0.5 * math.log(2.0 * math.pi)


def _np_constants():
    S = np.zeros((_DISC, _NSEG), dtype=np.float32)
    starts = np.cumsum([0] + list(_ACTION_SIZES))
    for s, (c0, c1) in enumerate(zip(starts[:-1], starts[1:])):
        S[c0:c1, s] = 1.0
    S16 = np.zeros((16, _DISC), dtype=np.float32)
    S16[:_NSEG, :] = S.T
    E_cont = np.zeros((155, 23), dtype=np.float32)
    for i in range(23):
        E_cont[132 + i, i] = 1.0
    E_agent = np.zeros((161, 6), dtype=np.float32)
    for i in range(6):
        E_agent[155 + i, i] = 1.0
    return S, S16, E_cont, E_agent, starts


_S_NP, _S16_NP, _ECONT_NP, _EAGENT_NP, _STARTS = _np_constants()


def _sc_argmax(action):
    B, ACT_W = action.shape
    NC, NS, L = 2, 16, 16
    NW = NC * NS
    rows_per_w = B // NW
    ntiles = rows_per_w // L

    @functools.partial(
        pl.kernel,
        out_type=jax.ShapeDtypeStruct((B, 16), jnp.float32),
        mesh=plsc.VectorSubcoreMesh(core_axis_name="c", subcore_axis_name="s"),
        scratch_types=[
            pltpu.VMEM((rows_per_w, ACT_W), jnp.float32),
            pltpu.VMEM((rows_per_w, 16), jnp.float32),
        ],
        compiler_params=pltpu.CompilerParams(use_tc_tiling_on_sc=False,
                                             needs_layout_passes=False),
    )
    def k(action_hbm, out_hbm, abuf, obuf):
        wid = lax.axis_index("s") * NC + lax.axis_index("c")
        base = wid * rows_per_w
        pltpu.sync_copy(action_hbm.at[pl.ds(base, rows_per_w)], abuf)
        lanes = lax.iota(jnp.int32, L)

        def tile_body(t, carry):
            rows = t * L + lanes
            zero = jnp.zeros((L,), jnp.float32)
            for s in range(_NSEG):
                c0, c1 = int(_STARTS[s]), int(_STARTS[s + 1])
                m = plsc.load_gather(abuf, [rows, jnp.full((L,), c0, jnp.int32)])
                bi = zero + float(c0)
                for j in range(c0 + 1, c1):
                    v = plsc.load_gather(
                        abuf, [rows, jnp.full((L,), j, jnp.int32)])
                    better = v > m
                    m = jnp.where(better, v, m)
                    bi = jnp.where(better, zero + float(j), bi)
                plsc.store_scatter(obuf, [rows, jnp.full((L,), s, jnp.int32)], bi)
            plsc.store_scatter(obuf, [rows, jnp.full((L,), 15, jnp.int32)], zero)
            return carry

        lax.fori_loop(0, ntiles, tile_body, 0)
        pltpu.sync_copy(obuf, out_hbm.at[pl.ds(base, rows_per_w)])

    return k(action)


def _policy_kernel(state_ref, cont_ref, aact_ref, idx_ref, w1t_ref, b1_ref,
                   wdt_ref, bd_ref, wcmt_ref, bcm_ref, wcst_ref, bcs_ref,
                   wa1t_ref, ba1_ref, wamt_ref, bam_ref, wast_ref, bas_ref,
                   s_ref, s16_ref, out_ref):
    f32 = jnp.float32
    bf16 = jnp.bfloat16
    x = state_ref[...]
    xb = x.astype(bf16)

    idxfull = jnp.dot(idx_ref[...], s16_ref[...], preferred_element_type=f32)
    iota = jax.lax.broadcasted_iota(jnp.int32, (idxfull.shape[0], _DISC), 1)
    oh = (iota == idxfull.astype(jnp.int32)).astype(f32)

    h = jnp.dot(xb, w1t_ref[...], preferred_element_type=f32) + b1_ref[...]
    h = jnp.where(h >= 0.0, h, 0.01 * h)
    hb = h.astype(bf16)

    logits = jnp.dot(hb, wdt_ref[...], preferred_element_type=f32) + bd_ref[...]
    mean = jnp.clip(jnp.dot(h, wcmt_ref[...], preferred_element_type=f32)
                    + bcm_ref[...], -1.0, 1.0)
    logstd = jnp.clip(jnp.dot(h, wcst_ref[...], preferred_element_type=f32)
                      + bcs_ref[...], 0.0, 1.0)

    continuous = cont_ref[...]

    gmax = jnp.max(logits, axis=1, keepdims=True)
    e = jnp.exp(logits - gmax)
    segsum = jnp.dot(e, s_ref[...], preferred_element_type=f32)
    lse = jnp.log(segsum) + gmax

    chosen = jnp.dot(logits * oh, s_ref[...], preferred_element_type=f32)
    seg_lp = chosen - lse

    cont_lp = (-(continuous - mean) ** 2 * (0.5 * jnp.exp(-2.0 * logstd))
               - logstd - _HALF_LOG_2PI)

    h2 = jnp.dot(xb, wa1t_ref[...], preferred_element_type=f32) + ba1_ref[...]
    h2 = jnp.where(h2 >= 0.0, h2, 0.01 * h2)
    m2 = jnp.clip(jnp.dot(h2, wamt_ref[...], preferred_element_type=f32)
                  + bam_ref[...], -1.0, 1.0)
    ls2 = jnp.clip(jnp.dot(h2, wast_ref[...], preferred_element_type=f32)
                   + bas_ref[...], 0.0, 1.0)
    aact = aact_ref[...]
    agent_lp = (-(aact - m2) ** 2 * (0.5 * jnp.exp(-2.0 * ls2))
                - ls2 - _HALF_LOG_2PI)

    out_ref[...] = jnp.concatenate([seg_lp, cont_lp, agent_lp], axis=1)


def _run_tc(state, action, idx16, W1, b1, Wd, bd, Wc, bc, Wa1, ba1, Wa2, ba2,
            interpret=False):
    B = state.shape[0]
    cont_in = action[:, 132:155]
    aact_in = state[:, 155:161]
    w1t = W1.T.astype(jnp.bfloat16)
    bd2 = bd[None, :]
    wdt = Wd.T.astype(jnp.bfloat16)
    wcmt = Wc[:23].T
    bcm = bc[None, :23]
    wcst = Wc[23:].T
    bcs = bc[None, 23:]
    wa1t = jnp.zeros((161, 128), jnp.bfloat16).at[:155, :].set(
        Wa1.T.astype(jnp.bfloat16))
    wamt = Wa2[:6].T
    bam = ba2[None, :6]
    wast = Wa2[6:].T
    bas = ba2[None, 6:]
    S = jnp.asarray(_S_NP)
    S16 = jnp.asarray(_S16_NP)

    grid = (B // _BB,)
    row = lambda i: (i, 0)
    rep = lambda i: (0, 0)
    full = lambda a: pl.BlockSpec(a.shape, rep)
    out = pl.pallas_call(
        _policy_kernel,
        grid=grid,
        in_specs=[
            pl.BlockSpec((_BB, 161), row),
            pl.BlockSpec((_BB, 23), row),
            pl.BlockSpec((_BB, 6), row),
            pl.BlockSpec((_BB, 16), row),
            full(w1t), full(b1[None, :]), full(wdt), full(bd2),
            full(wcmt), full(bcm), full(wcst), full(bcs),
            full(wa1t), full(ba1[None, :]),
            full(wamt), full(bam), full(wast), full(bas),
            full(S), full(S16),
        ],
        out_specs=pl.BlockSpec((_BB, 44), row),
        out_shape=jax.ShapeDtypeStruct((B, 44), jnp.float32),
        compiler_params=pltpu.CompilerParams(
            dimension_semantics=("parallel",)),
        interpret=interpret,
    )(state, cont_in, aact_in, idx16, w1t, b1[None, :], wdt, bd2, wcmt, bcm,
      wcst, bcs, wa1t, ba1[None, :], wamt, bam, wast, bas, S, S16)
    return out


@jax.jit
def _run(state, action, W1, b1, Wd, bd, Wc, bc, Wa1, ba1, Wa2, ba2):
    idx16 = _sc_argmax(action)
    return _run_tc(state, action, idx16, W1, b1, Wd, bd, Wc, bc,
                   Wa1, ba1, Wa2, ba2)


def kernel(state, action, W1, b1, Wd, bd, Wc, bc, Wa1, ba1, Wa2, ba2):
    return _run(state, action, W1, b1, Wd, bd, Wc, bc, Wa1, ba1, Wa2, ba2)


# speedup vs baseline: 1.6113x; 1.0029x over previous
import functools
import math

import jax
import jax.numpy as jnp
import numpy as np
from jax import lax
from jax.experimental import pallas as pl
from jax.experimental.pallas import tpu as pltpu
from jax.experimental.pallas import tpu_sc as plsc

_ACTION_SIZES = (5, 2, 4, 3, 2, 9, 2, 32, 35, 7, 2, 21, 2, 3, 3)
_NSEG = len(_ACTION_SIZES)
_DISC = 132
_BB = 2048
_HALF_LOG_2PI = 0.5 * math.log(2.0 * math.pi)


def _np_constants():
    S = np.zeros((_DISC, _NSEG), dtype=np.float32)
    starts = np.cumsum([0] + list(_ACTION_SIZES))
    for s, (c0, c1) in enumerate(zip(starts[:-1], starts[1:])):
        S[c0:c1, s] = 1.0
    S16 = np.zeros((16, _DISC), dtype=np.float32)
    S16[:_NSEG, :] = S.T
    E_cont = np.zeros((155, 23), dtype=np.float32)
    for i in range(23):
        E_cont[132 + i, i] = 1.0
    E_agent = np.zeros((161, 6), dtype=np.float32)
    for i in range(6):
        E_agent[155 + i, i] = 1.0
    return S, S16, E_cont, E_agent, starts


_S_NP, _S16_NP, _ECONT_NP, _EAGENT_NP, _STARTS = _np_constants()


def _sc_argmax(action):
    B, ACT_W = action.shape
    NC, NS, L = 2, 16, 16
    NW = NC * NS
    rows_per_w = B // NW
    ntiles = rows_per_w // L

    @functools.partial(
        pl.kernel,
        out_type=jax.ShapeDtypeStruct((B, 16), jnp.float32),
        mesh=plsc.VectorSubcoreMesh(core_axis_name="c", subcore_axis_name="s"),
        scratch_types=[
            pltpu.VMEM((rows_per_w, ACT_W), jnp.float32),
            pltpu.VMEM((rows_per_w, 16), jnp.float32),
        ],
        compiler_params=pltpu.CompilerParams(use_tc_tiling_on_sc=False,
                                             needs_layout_passes=False),
    )
    def k(action_hbm, out_hbm, abuf, obuf):
        wid = lax.axis_index("s") * NC + lax.axis_index("c")
        base = wid * rows_per_w
        pltpu.sync_copy(action_hbm.at[pl.ds(base, rows_per_w)], abuf)
        lanes = lax.iota(jnp.int32, L)

        def process(rows):
            zero = jnp.zeros((L,), jnp.float32)
            for s in range(_NSEG):
                c0, c1 = int(_STARTS[s]), int(_STARTS[s + 1])
                m = plsc.load_gather(abuf, [rows, jnp.full((L,), c0, jnp.int32)])
                bi = zero + float(c0)
                for j in range(c0 + 1, c1):
                    v = plsc.load_gather(
                        abuf, [rows, jnp.full((L,), j, jnp.int32)])
                    better = v > m
                    m = jnp.where(better, v, m)
                    bi = jnp.where(better, zero + float(j), bi)
                plsc.store_scatter(obuf, [rows, jnp.full((L,), s, jnp.int32)], bi)
            plsc.store_scatter(obuf, [rows, jnp.full((L,), 15, jnp.int32)], zero)

        def tile_body(t, carry):
            process((2 * t) * L + lanes)
            process((2 * t + 1) * L + lanes)
            return carry

        lax.fori_loop(0, ntiles // 2, tile_body, 0)
        pltpu.sync_copy(obuf, out_hbm.at[pl.ds(base, rows_per_w)])

    return k(action)


def _policy_kernel(state_ref, cont_ref, aact_ref, idx_ref, w1t_ref, b1_ref,
                   wdt_ref, bd_ref, wcmt_ref, bcm_ref, wcst_ref, bcs_ref,
                   wa1t_ref, ba1_ref, wamt_ref, bam_ref, wast_ref, bas_ref,
                   s_ref, s16_ref, out_ref):
    f32 = jnp.float32
    bf16 = jnp.bfloat16
    x = state_ref[...]
    xb = x.astype(bf16)

    idxfull = jnp.dot(idx_ref[...], s16_ref[...], preferred_element_type=f32)
    iota = jax.lax.broadcasted_iota(jnp.int32, (idxfull.shape[0], _DISC), 1)
    oh = (iota == idxfull.astype(jnp.int32)).astype(f32)

    h = jnp.dot(xb, w1t_ref[...], preferred_element_type=f32) + b1_ref[...]
    h = jnp.where(h >= 0.0, h, 0.01 * h)
    hb = h.astype(bf16)

    logits = jnp.dot(hb, wdt_ref[...], preferred_element_type=f32) + bd_ref[...]
    mean = jnp.clip(jnp.dot(h, wcmt_ref[...], preferred_element_type=f32)
                    + bcm_ref[...], -1.0, 1.0)
    logstd = jnp.clip(jnp.dot(h, wcst_ref[...], preferred_element_type=f32)
                      + bcs_ref[...], 0.0, 1.0)

    continuous = cont_ref[...]

    gmax = jnp.max(logits, axis=1, keepdims=True)
    e = jnp.exp(logits - gmax)
    segsum = jnp.dot(e, s_ref[...], preferred_element_type=f32)
    lse = jnp.log(segsum) + gmax

    chosen = jnp.dot(logits * oh, s_ref[...], preferred_element_type=f32)
    seg_lp = chosen - lse

    cont_lp = (-(continuous - mean) ** 2 * (0.5 * jnp.exp(-2.0 * logstd))
               - logstd - _HALF_LOG_2PI)

    h2 = jnp.dot(xb, wa1t_ref[...], preferred_element_type=f32) + ba1_ref[...]
    h2 = jnp.where(h2 >= 0.0, h2, 0.01 * h2)
    m2 = jnp.clip(jnp.dot(h2, wamt_ref[...], preferred_element_type=f32)
                  + bam_ref[...], -1.0, 1.0)
    ls2 = jnp.clip(jnp.dot(h2, wast_ref[...], preferred_element_type=f32)
                   + bas_ref[...], 0.0, 1.0)
    aact = aact_ref[...]
    agent_lp = (-(aact - m2) ** 2 * (0.5 * jnp.exp(-2.0 * ls2))
                - ls2 - _HALF_LOG_2PI)

    out_ref[...] = jnp.concatenate([seg_lp, cont_lp, agent_lp], axis=1)


def _run_tc(state, action, idx16, W1, b1, Wd, bd, Wc, bc, Wa1, ba1, Wa2, ba2,
            interpret=False):
    B = state.shape[0]
    cont_in = action[:, 132:155]
    aact_in = state[:, 155:161]
    w1t = W1.T.astype(jnp.bfloat16)
    bd2 = bd[None, :]
    wdt = Wd.T.astype(jnp.bfloat16)
    wcmt = Wc[:23].T
    bcm = bc[None, :23]
    wcst = Wc[23:].T
    bcs = bc[None, 23:]
    wa1t = jnp.zeros((161, 128), jnp.bfloat16).at[:155, :].set(
        Wa1.T.astype(jnp.bfloat16))
    wamt = Wa2[:6].T
    bam = ba2[None, :6]
    wast = Wa2[6:].T
    bas = ba2[None, 6:]
    S = jnp.asarray(_S_NP)
    S16 = jnp.asarray(_S16_NP)

    grid = (B // _BB,)
    row = lambda i: (i, 0)
    rep = lambda i: (0, 0)
    full = lambda a: pl.BlockSpec(a.shape, rep)
    out = pl.pallas_call(
        _policy_kernel,
        grid=grid,
        in_specs=[
            pl.BlockSpec((_BB, 161), row),
            pl.BlockSpec((_BB, 23), row),
            pl.BlockSpec((_BB, 6), row),
            pl.BlockSpec((_BB, 16), row),
            full(w1t), full(b1[None, :]), full(wdt), full(bd2),
            full(wcmt), full(bcm), full(wcst), full(bcs),
            full(wa1t), full(ba1[None, :]),
            full(wamt), full(bam), full(wast), full(bas),
            full(S), full(S16),
        ],
        out_specs=pl.BlockSpec((_BB, 44), row),
        out_shape=jax.ShapeDtypeStruct((B, 44), jnp.float32),
        compiler_params=pltpu.CompilerParams(
            dimension_semantics=("parallel",)),
        interpret=interpret,
    )(state, cont_in, aact_in, idx16, w1t, b1[None, :], wdt, bd2, wcmt, bcm,
      wcst, bcs, wa1t, ba1[None, :], wamt, bam, wast, bas, S, S16)
    return out


@jax.jit
def _run(state, action, W1, b1, Wd, bd, Wc, bc, Wa1, ba1, Wa2, ba2):
    idx16 = _sc_argmax(action)
    return _run_tc(state, action, idx16, W1, b1, Wd, bd, Wc, bc,
                   Wa1, ba1, Wa2, ba2)


def kernel(state, action, W1, b1, Wd, bd, Wc, bc, Wa1, ba1, Wa2, ba2):
    return _run(state, action, W1, b1, Wd, bd, Wc, bc, Wa1, ba1, Wa2, ba2)
